# baseline (speedup 1.0000x reference)
import functools
import jax
import jax.numpy as jnp
from jax import lax
from jax.experimental import pallas as pl
from jax.experimental.pallas import tpu as pltpu
from jax.experimental.pallas import tpu_sc as plsc

NUM_NODE = 100000
DIM = 128
K = 4
DIM_IT = 64
B = 4096
L = 20
PAD_L = 24

NC = 2
NS = 16
NW = NC * NS
BPW = B // NW
CB = 4
CL = CB * L
CLP = CB * PAD_L
NCH = BPW // CB
CPW = 2
WB = CB * CPW
NWV = BPW // WB
RW = CL * CPW


def _sc_emb_body(emb_hbm, it_idx_hbm, nb_idx_hbm,
                 sum_t_hbm, sum_n_hbm,
                 it_idx_v, nb_idx_v,
                 be0, be1, bn0, bn1, ws0, ws1, wn0, wn1,
                 se0, se1, ss0, ss1, sn0, sn1, sv0, sv1):
    c = lax.axis_index("c")
    s = lax.axis_index("s")
    w = c * NS + s
    base_b = w * BPW

    pltpu.sync_copy(it_idx_hbm.at[w], it_idx_v)
    pltpu.sync_copy(nb_idx_hbm.at[w], nb_idx_v)

    def fire(idx_v, wv, buf, se):
        for q in range(CPW):
            pltpu.async_copy(emb_hbm.at[idx_v.at[wv * CPW + q]],
                             buf.at[pl.ds(q * CL, CL)], se)

    def drain(idx_v, wv, buf, se):
        for q in range(CPW):
            pltpu.make_async_copy(emb_hbm.at[idx_v.at[wv * CPW + q]],
                                  buf.at[pl.ds(q * CL, CL)], se).wait()

    def process(idx_v, sum_hbm, wv, buf, se, ws, ss):
        drain(idx_v, wv, buf, se)

        @pl.when(wv >= 2)
        def _():
            pltpu.make_async_copy(ws, sum_hbm.at[pl.ds(base_b, WB)], ss).wait()

        def bb(bi, cc):
            r0 = bi * L
            accs = [buf[r0, pl.ds(g * 16, 16)] for g in range(8)]
            for l in range(1, L):
                for g in range(8):
                    accs[g] = accs[g] + buf[r0 + l, pl.ds(g * 16, 16)]
            for g in range(8):
                ws[bi, pl.ds(g * 16, 16)] = accs[g]
            return cc

        lax.fori_loop(0, WB, bb, 0)
        pltpu.async_copy(ws, sum_hbm.at[pl.ds(base_b + wv * WB, WB)], ss)

        @pl.when(wv + 2 < NWV)
        def _():
            fire(idx_v, wv + 2, buf, se)

    fire(it_idx_v, 0, be0, se0)
    fire(nb_idx_v, 0, bn0, sn0)
    fire(it_idx_v, 1, be1, se1)
    fire(nb_idx_v, 1, bn1, sn1)

    def body(h, carry):
        for par, bt, se, wst, sst, bn, sn, wsn, ssn in (
                (0, be0, se0, ws0, ss0, bn0, sn0, wn0, sv0),
                (1, be1, se1, ws1, ss1, bn1, sn1, wn1, sv1)):
            wv = 2 * h + par
            process(it_idx_v, sum_t_hbm, wv, bt, se, wst, sst)
            process(nb_idx_v, sum_n_hbm, wv, bn, sn, wsn, ssn)
        return carry

    lax.fori_loop(0, NWV // 2, body, 0)
    pltpu.make_async_copy(ws0, sum_t_hbm.at[pl.ds(base_b, WB)], ss0).wait()
    pltpu.make_async_copy(ws1, sum_t_hbm.at[pl.ds(base_b, WB)], ss1).wait()
    pltpu.make_async_copy(wn0, sum_n_hbm.at[pl.ds(base_b, WB)], sv0).wait()
    pltpu.make_async_copy(wn1, sum_n_hbm.at[pl.ds(base_b, WB)], sv1).wait()


def _sc_emb(emb, it_idx, nb_idx):
    mesh = plsc.VectorSubcoreMesh(core_axis_name="c", subcore_axis_name="s")
    f = pl.kernel(
        _sc_emb_body,
        out_type=[
            jax.ShapeDtypeStruct((B, DIM), jnp.float32),
            jax.ShapeDtypeStruct((B, DIM), jnp.float32),
        ],
        mesh=mesh,
        scratch_types=[
            pltpu.VMEM((NCH, CL), jnp.int32),
            pltpu.VMEM((NCH, CL), jnp.int32),
            pltpu.VMEM((RW, DIM), jnp.float32),
            pltpu.VMEM((RW, DIM), jnp.float32),
            pltpu.VMEM((RW, DIM), jnp.float32),
            pltpu.VMEM((RW, DIM), jnp.float32),
            pltpu.VMEM((WB, DIM), jnp.float32),
            pltpu.VMEM((WB, DIM), jnp.float32),
            pltpu.VMEM((WB, DIM), jnp.float32),
            pltpu.VMEM((WB, DIM), jnp.float32),
            pltpu.SemaphoreType.DMA,
            pltpu.SemaphoreType.DMA,
            pltpu.SemaphoreType.DMA,
            pltpu.SemaphoreType.DMA,
            pltpu.SemaphoreType.DMA,
            pltpu.SemaphoreType.DMA,
            pltpu.SemaphoreType.DMA,
            pltpu.SemaphoreType.DMA,
        ],
    )
    return f(emb, it_idx, nb_idx)


def _sc_img_body(img_hbm, it_idx_pad_hbm, img_rows_hbm,
                 it_idx_pad_v, bi0, bi1,
                 si0, si1, sw0, sw1):
    c = lax.axis_index("c")
    s = lax.axis_index("s")
    w = c * NS + s
    base_b = w * BPW

    pltpu.sync_copy(it_idx_pad_hbm.at[w], it_idx_pad_v)

    def img_dst(ci):
        return img_rows_hbm.at[pl.ds((base_b + ci * CB) * PAD_L, CLP)]

    def img_fire(ci, buf, si):
        pltpu.async_copy(img_hbm.at[it_idx_pad_v.at[ci]], buf, si)

    img_fire(0, bi0, si0)
    img_fire(1, bi1, si1)

    def img_body(h, carry):
        for par, buf, si, sw in ((0, bi0, si0, sw0), (1, bi1, si1, sw1)):
            ci = 2 * h + par
            pltpu.make_async_copy(img_hbm.at[it_idx_pad_v.at[ci]], buf, si).wait()
            pltpu.async_copy(buf, img_dst(ci), sw)

            @pl.when(ci + 2 < NCH)
            def _():
                pltpu.make_async_copy(buf, img_dst(ci), sw).wait()
                img_fire(ci + 2, buf, si)
        return carry

    lax.fori_loop(0, NCH // 2, img_body, 0)
    pltpu.make_async_copy(bi0, img_dst(NCH - 2), sw0).wait()
    pltpu.make_async_copy(bi1, img_dst(NCH - 1), sw1).wait()


def _sc_img(img_bf, it_idx_pad):
    mesh = plsc.VectorSubcoreMesh(core_axis_name="c", subcore_axis_name="s")
    f = pl.kernel(
        _sc_img_body,
        out_type=[
            jax.ShapeDtypeStruct((B * PAD_L, DIM_IT), jnp.bfloat16),
        ],
        mesh=mesh,
        scratch_types=[
            pltpu.VMEM((NCH, CLP), jnp.int32),
            pltpu.VMEM((CLP, DIM_IT), jnp.bfloat16),
            pltpu.VMEM((CLP, DIM_IT), jnp.bfloat16),
            pltpu.SemaphoreType.DMA,
            pltpu.SemaphoreType.DMA,
            pltpu.SemaphoreType.DMA,
            pltpu.SemaphoreType.DMA,
        ],
        compiler_params=pltpu.CompilerParams(use_tc_tiling_on_sc=False),
    )
    return f(img_bf, it_idx_pad)[0]


BB1 = 512


def _meta_body(img_ref, sum_t_ref, sum_n_ref,
               wia_ref, bia_ref, wm_ref, bm_ref,
               w1p_ref, b1p_ref, a1_ref, w1o_ref, b1o_ref,
               w2p_ref, b2p_ref, a2_ref, w2o_ref, b2o_ref,
               m1_ref, m2_ref, acc1_ref, acc2_ref):
    f32 = jnp.float32
    img3 = img_ref[...].astype(f32).reshape(BB1, PAD_L, DIM_IT)
    sum_img = jnp.sum(img3[:, :L, :], axis=1)
    sum_auxi = (jnp.dot(sum_img, wia_ref[...], preferred_element_type=f32)
                + jnp.float32(L) * bia_ref[...])
    sum_t = sum_t_ref[...].astype(f32)
    sum_n = sum_n_ref[...].astype(f32)
    tembedi = (jnp.dot(sum_auxi, wm_ref[0:DIM, :], preferred_element_type=f32)
               + jnp.dot(sum_t, wm_ref[DIM:2 * DIM, :], preferred_element_type=f32)
               + jnp.dot(sum_n, wm_ref[2 * DIM:3 * DIM, :], preferred_element_type=f32)
               + bm_ref[...])

    def mlp(wp_ref, bp_ref, a_ref, wo_ref, bo_ref):
        h = jnp.dot(tembedi, wp_ref[...], preferred_element_type=f32) + bp_ref[...]
        a = a_ref[0, 0]
        h = jnp.where(h >= 0, h, a * h)
        m = jnp.dot(h, wo_ref[...], preferred_element_type=f32) + bo_ref[...]
        n = jnp.sqrt(jnp.sum(m * m, axis=-1, keepdims=True))
        return m / jnp.maximum(n, 1e-12)

    m1 = mlp(w1p_ref, b1p_ref, a1_ref, w1o_ref, b1o_ref)
    m2 = mlp(w2p_ref, b2p_ref, a2_ref, w2o_ref, b2o_ref)
    m1_ref[...] = m1
    m2_ref[...] = m2

    @pl.when(pl.program_id(0) == 0)
    def _():
        acc1_ref[...] = jnp.zeros_like(acc1_ref)
        acc2_ref[...] = jnp.zeros_like(acc2_ref)

    acc1_ref[...] += jnp.sum(m1, axis=0, keepdims=True)
    acc2_ref[...] += jnp.sum(m2, axis=0, keepdims=True)


def _tc_meta(img_rows, sum_t, sum_n, wia, bia, wm, bm,
             w1p, b1p, a1, w1o_p, b1o_p, w2p, b2p, a2, w2o, b2o):
    dk = DIM * K
    full = lambda shape: pl.BlockSpec(shape, lambda i: tuple(0 for _ in shape))
    return pl.pallas_call(
        _meta_body,
        grid=(B // BB1,),
        in_specs=[
            pl.BlockSpec((BB1 * PAD_L, DIM_IT), lambda i: (i, 0)),
            pl.BlockSpec((BB1, DIM), lambda i: (i, 0)),
            pl.BlockSpec((BB1, DIM), lambda i: (i, 0)),
            full((DIM_IT, DIM)), full((1, DIM)), full((3 * DIM, DIM)), full((1, DIM)),
            full((DIM, dk)), full((1, dk)), full((1, 1)), full((dk, dk)), full((1, dk)),
            full((DIM, dk)), full((1, dk)), full((1, 1)), full((dk, dk)), full((1, dk)),
        ],
        out_specs=[
            pl.BlockSpec((BB1, dk), lambda i: (i, 0)),
            pl.BlockSpec((BB1, dk), lambda i: (i, 0)),
            pl.BlockSpec((1, dk), lambda i: (0, 0)),
            pl.BlockSpec((1, dk), lambda i: (0, 0)),
        ],
        out_shape=[
            jax.ShapeDtypeStruct((B, dk), jnp.float32),
            jax.ShapeDtypeStruct((B, dk), jnp.float32),
            jax.ShapeDtypeStruct((1, dk), jnp.float32),
            jax.ShapeDtypeStruct((1, dk), jnp.float32),
        ],
    )(img_rows, sum_t, sum_n, wia, bia, wm, bm,
      w1p, b1p, a1, w1o_p, b1o_p, w2p, b2p, a2, w2o, b2o)


BB2 = 256


def _apply_body(m1_ref, m2_ref, acc1_ref, acc2_ref, img_ref, wia_ref, bia_ref,
                out_ref):
    f32 = jnp.float32
    inv_b = jnp.float32(1.0 / B)
    m1 = m1_ref[...] + acc1_ref[...] * inv_b
    m2 = m2_ref[...] + acc2_ref[...] * inv_b

    w1 = []
    for k in range(K):
        sl = m1[:, k * DIM:(k + 1) * DIM]
        mx = jnp.max(sl, axis=-1, keepdims=True)
        e = jnp.exp(sl - mx)
        w1.append(e / jnp.sum(e, axis=-1, keepdims=True))

    s2 = [m2[:, k * DIM:(k + 1) * DIM] for k in range(K)]
    mx2 = jnp.maximum(jnp.maximum(s2[0], s2[1]), jnp.maximum(s2[2], s2[3]))
    e2 = [jnp.exp(x - mx2) for x in s2]
    den = e2[0] + e2[1] + e2[2] + e2[3]
    w2 = [x / den for x in e2]

    img = img_ref[...].astype(f32)
    auxi = jnp.dot(img, wia_ref[...], preferred_element_type=f32) + bia_ref[...]
    auxi3 = auxi.reshape(BB2, PAD_L, DIM)

    t3 = jnp.zeros((BB2, PAD_L, DIM), f32)
    for k in range(K):
        t1 = jnp.sum(auxi3 * w1[k][:, None, :], axis=-1)
        t3 = t3 + t1[:, :, None] * w2[k][:, None, :]
    out_ref[...] = t3[:, :L, :]


def _tc_apply(m1, m2, acc1, acc2, img_rows, wia, bia):
    dk = DIM * K
    return pl.pallas_call(
        _apply_body,
        grid=(B // BB2,),
        in_specs=[
            pl.BlockSpec((BB2, dk), lambda i: (i, 0)),
            pl.BlockSpec((BB2, dk), lambda i: (i, 0)),
            pl.BlockSpec((1, dk), lambda i: (0, 0)),
            pl.BlockSpec((1, dk), lambda i: (0, 0)),
            pl.BlockSpec((BB2 * PAD_L, DIM_IT), lambda i: (i, 0)),
            pl.BlockSpec((DIM_IT, DIM), lambda i: (0, 0)),
            pl.BlockSpec((1, DIM), lambda i: (0, 0)),
        ],
        out_specs=pl.BlockSpec((BB2, L, DIM), lambda i: (i, 0, 0)),
        out_shape=jax.ShapeDtypeStruct((B, L, DIM), jnp.float32),
    )(m1, m2, acc1, acc2, img_rows, wia, bia)


def kernel(emb, img_emb, W_img2id, b_img2id, W_meta, b_meta,
           W1_pre, b1_pre, a1, W1_out, b1_out,
           W2_pre, b2_pre, a2, W2_out, b2_out,
           item_seq, neighbor_seq):
    dk = DIM * K
    it32 = item_seq.astype(jnp.int32)
    it_idx = it32.reshape(NW, NCH, CL)
    it_idx_pad = jnp.pad(it32, ((0, 0), (0, PAD_L - L))).reshape(NW, NCH, CLP)
    nb_idx = neighbor_seq.astype(jnp.int32).reshape(NW, NCH, CL)
    img_bf = img_emb.astype(jnp.bfloat16)

    sum_t, sum_n = _sc_emb(emb, it_idx, nb_idx)
    img_rows = _sc_img(img_bf, it_idx_pad)

    w1o_p = W1_out.reshape(dk, DIM, K).transpose(0, 2, 1).reshape(dk, dk)
    b1o_p = b1_out.reshape(DIM, K).T.reshape(1, dk)

    m1, m2, acc1, acc2 = _tc_meta(
        img_rows, sum_t, sum_n,
        W_img2id, b_img2id.reshape(1, DIM), W_meta, b_meta.reshape(1, DIM),
        W1_pre, b1_pre.reshape(1, dk), a1.reshape(1, 1), w1o_p, b1o_p,
        W2_pre, b2_pre.reshape(1, dk), a2.reshape(1, 1), W2_out, b2_out.reshape(1, dk))

    return _tc_apply(m1, m2, acc1, acc2, img_rows,
                     W_img2id, b_img2id.reshape(1, DIM))


# speedup vs baseline: 1.0031x; 1.0031x over previous
import functools
import jax
import jax.numpy as jnp
from jax import lax
from jax.experimental import pallas as pl
from jax.experimental.pallas import tpu as pltpu
from jax.experimental.pallas import tpu_sc as plsc

NUM_NODE = 100000
DIM = 128
K = 4
DIM_IT = 64
B = 4096
L = 20
PAD_L = 24

NC = 2
NS = 16
NW = NC * NS
BPW = B // NW
CB = 4
CL = CB * L
CLP = CB * PAD_L
NCH = BPW // CB
CPW = 2
WB = CB * CPW
NWV = BPW // WB
RW = CL * CPW


def _sc_emb_body(emb_hbm, it_idx_hbm, nb_idx_hbm,
                 sum_t_hbm, sum_n_hbm,
                 it_idx_v, nb_idx_v,
                 be0, be1, bn0, bn1, ws0, ws1, wn0, wn1,
                 se0, se1, ss0, ss1, sn0, sn1, sv0, sv1):
    c = lax.axis_index("c")
    s = lax.axis_index("s")
    w = c * NS + s
    base_b = w * BPW

    pltpu.sync_copy(it_idx_hbm.at[w], it_idx_v)
    pltpu.sync_copy(nb_idx_hbm.at[w], nb_idx_v)

    def fire(idx_v, wv, buf, se):
        for q in range(CPW):
            pltpu.async_copy(emb_hbm.at[idx_v.at[wv * CPW + q]],
                             buf.at[pl.ds(q * CL, CL)], se)

    def drain(idx_v, wv, buf, se):
        for q in range(CPW):
            pltpu.make_async_copy(emb_hbm.at[idx_v.at[wv * CPW + q]],
                                  buf.at[pl.ds(q * CL, CL)], se).wait()

    def process(idx_v, sum_hbm, wv, buf, se, ws, ss):
        drain(idx_v, wv, buf, se)

        @pl.when(wv >= 2)
        def _():
            pltpu.make_async_copy(ws, sum_hbm.at[pl.ds(base_b, WB)], ss).wait()

        def bb(bi, cc):
            r0 = bi * L
            accs = [buf[r0, pl.ds(g * 16, 16)] for g in range(8)]
            for l in range(1, L):
                for g in range(8):
                    accs[g] = accs[g] + buf[r0 + l, pl.ds(g * 16, 16)]
            for g in range(8):
                ws[bi, pl.ds(g * 16, 16)] = accs[g]
            return cc

        lax.fori_loop(0, WB, bb, 0)
        pltpu.async_copy(ws, sum_hbm.at[pl.ds(base_b + wv * WB, WB)], ss)

        @pl.when(wv + 2 < NWV)
        def _():
            fire(idx_v, wv + 2, buf, se)

    fire(it_idx_v, 0, be0, se0)
    fire(nb_idx_v, 0, bn0, sn0)
    fire(it_idx_v, 1, be1, se1)
    fire(nb_idx_v, 1, bn1, sn1)

    def body(h, carry):
        for par, bt, se, wst, sst, bn, sn, wsn, ssn in (
                (0, be0, se0, ws0, ss0, bn0, sn0, wn0, sv0),
                (1, be1, se1, ws1, ss1, bn1, sn1, wn1, sv1)):
            wv = 2 * h + par
            process(it_idx_v, sum_t_hbm, wv, bt, se, wst, sst)
            process(nb_idx_v, sum_n_hbm, wv, bn, sn, wsn, ssn)
        return carry

    lax.fori_loop(0, NWV // 2, body, 0)
    pltpu.make_async_copy(ws0, sum_t_hbm.at[pl.ds(base_b, WB)], ss0).wait()
    pltpu.make_async_copy(ws1, sum_t_hbm.at[pl.ds(base_b, WB)], ss1).wait()
    pltpu.make_async_copy(wn0, sum_n_hbm.at[pl.ds(base_b, WB)], sv0).wait()
    pltpu.make_async_copy(wn1, sum_n_hbm.at[pl.ds(base_b, WB)], sv1).wait()


def _sc_emb(emb, it_idx, nb_idx):
    mesh = plsc.VectorSubcoreMesh(core_axis_name="c", subcore_axis_name="s")
    f = pl.kernel(
        _sc_emb_body,
        out_type=[
            jax.ShapeDtypeStruct((B, DIM), jnp.float32),
            jax.ShapeDtypeStruct((B, DIM), jnp.float32),
        ],
        mesh=mesh,
        scratch_types=[
            pltpu.VMEM((NCH, CL), jnp.int32),
            pltpu.VMEM((NCH, CL), jnp.int32),
            pltpu.VMEM((RW, DIM), jnp.float32),
            pltpu.VMEM((RW, DIM), jnp.float32),
            pltpu.VMEM((RW, DIM), jnp.float32),
            pltpu.VMEM((RW, DIM), jnp.float32),
            pltpu.VMEM((WB, DIM), jnp.float32),
            pltpu.VMEM((WB, DIM), jnp.float32),
            pltpu.VMEM((WB, DIM), jnp.float32),
            pltpu.VMEM((WB, DIM), jnp.float32),
            pltpu.SemaphoreType.DMA,
            pltpu.SemaphoreType.DMA,
            pltpu.SemaphoreType.DMA,
            pltpu.SemaphoreType.DMA,
            pltpu.SemaphoreType.DMA,
            pltpu.SemaphoreType.DMA,
            pltpu.SemaphoreType.DMA,
            pltpu.SemaphoreType.DMA,
        ],
    )
    return f(emb, it_idx, nb_idx)


def _sc_img_body(img_hbm, it_idx_pad_hbm, img_rows_hbm,
                 it_idx_pad_v, bi0, bi1,
                 si0, si1, sw0, sw1):
    c = lax.axis_index("c")
    s = lax.axis_index("s")
    w = c * NS + s
    base_b = w * BPW

    pltpu.sync_copy(it_idx_pad_hbm.at[w], it_idx_pad_v)

    def img_dst(ci):
        return img_rows_hbm.at[pl.ds((base_b + ci * CB) * PAD_L, CLP)]

    def img_fire(ci, buf, si):
        pltpu.async_copy(img_hbm.at[it_idx_pad_v.at[ci]], buf, si)

    img_fire(0, bi0, si0)
    img_fire(1, bi1, si1)

    def img_body(h, carry):
        for par, buf, si, sw in ((0, bi0, si0, sw0), (1, bi1, si1, sw1)):
            ci = 2 * h + par
            pltpu.make_async_copy(img_hbm.at[it_idx_pad_v.at[ci]], buf, si).wait()
            pltpu.async_copy(buf, img_dst(ci), sw)

            @pl.when(ci + 2 < NCH)
            def _():
                pltpu.make_async_copy(buf, img_dst(ci), sw).wait()
                img_fire(ci + 2, buf, si)
        return carry

    lax.fori_loop(0, NCH // 2, img_body, 0)
    pltpu.make_async_copy(bi0, img_dst(NCH - 2), sw0).wait()
    pltpu.make_async_copy(bi1, img_dst(NCH - 1), sw1).wait()


def _sc_img(img_bf, it_idx_pad):
    mesh = plsc.VectorSubcoreMesh(core_axis_name="c", subcore_axis_name="s")
    f = pl.kernel(
        _sc_img_body,
        out_type=[
            jax.ShapeDtypeStruct((B * PAD_L, DIM_IT), jnp.bfloat16),
        ],
        mesh=mesh,
        scratch_types=[
            pltpu.VMEM((NCH, CLP), jnp.int32),
            pltpu.VMEM((CLP, DIM_IT), jnp.bfloat16),
            pltpu.VMEM((CLP, DIM_IT), jnp.bfloat16),
            pltpu.SemaphoreType.DMA,
            pltpu.SemaphoreType.DMA,
            pltpu.SemaphoreType.DMA,
            pltpu.SemaphoreType.DMA,
        ],
        compiler_params=pltpu.CompilerParams(use_tc_tiling_on_sc=False),
    )
    return f(img_bf, it_idx_pad)[0]


BB1 = 512


def _meta_body(img_ref, sum_t_ref, sum_n_ref,
               wia_ref, bia_ref, wm_ref, bm_ref,
               w1p_ref, b1p_ref, a1_ref, w1o_ref, b1o_ref,
               w2p_ref, b2p_ref, a2_ref, w2o_ref, b2o_ref,
               m1_ref, m2_ref, acc1_ref, acc2_ref):
    f32 = jnp.float32
    img3 = img_ref[...].astype(f32).reshape(BB1, PAD_L, DIM_IT)
    sum_img = jnp.sum(img3[:, :L, :], axis=1)
    sum_auxi = (jnp.dot(sum_img, wia_ref[...], preferred_element_type=f32)
                + jnp.float32(L) * bia_ref[...])
    sum_t = sum_t_ref[...].astype(f32)
    sum_n = sum_n_ref[...].astype(f32)
    tembedi = (jnp.dot(sum_auxi, wm_ref[0:DIM, :], preferred_element_type=f32)
               + jnp.dot(sum_t, wm_ref[DIM:2 * DIM, :], preferred_element_type=f32)
               + jnp.dot(sum_n, wm_ref[2 * DIM:3 * DIM, :], preferred_element_type=f32)
               + bm_ref[...])

    def mlp(wp_ref, bp_ref, a_ref, wo_ref, bo_ref):
        h = jnp.dot(tembedi, wp_ref[...], preferred_element_type=f32) + bp_ref[...]
        a = a_ref[0, 0]
        h = jnp.where(h >= 0, h, a * h)
        m = jnp.dot(h, wo_ref[...], preferred_element_type=f32) + bo_ref[...]
        n = jnp.sqrt(jnp.sum(m * m, axis=-1, keepdims=True))
        return m / jnp.maximum(n, 1e-12)

    m1 = mlp(w1p_ref, b1p_ref, a1_ref, w1o_ref, b1o_ref)
    m2 = mlp(w2p_ref, b2p_ref, a2_ref, w2o_ref, b2o_ref)
    m1_ref[...] = m1
    m2_ref[...] = m2

    @pl.when(pl.program_id(0) == 0)
    def _():
        acc1_ref[...] = jnp.zeros_like(acc1_ref)
        acc2_ref[...] = jnp.zeros_like(acc2_ref)

    acc1_ref[...] += jnp.sum(m1, axis=0, keepdims=True)
    acc2_ref[...] += jnp.sum(m2, axis=0, keepdims=True)


def _tc_meta(img_rows, sum_t, sum_n, wia, bia, wm, bm,
             w1p, b1p, a1, w1o_p, b1o_p, w2p, b2p, a2, w2o, b2o):
    dk = DIM * K
    full = lambda shape: pl.BlockSpec(shape, lambda i: tuple(0 for _ in shape))
    return pl.pallas_call(
        _meta_body,
        grid=(B // BB1,),
        in_specs=[
            pl.BlockSpec((BB1 * PAD_L, DIM_IT), lambda i: (i, 0)),
            pl.BlockSpec((BB1, DIM), lambda i: (i, 0)),
            pl.BlockSpec((BB1, DIM), lambda i: (i, 0)),
            full((DIM_IT, DIM)), full((1, DIM)), full((3 * DIM, DIM)), full((1, DIM)),
            full((DIM, dk)), full((1, dk)), full((1, 1)), full((dk, dk)), full((1, dk)),
            full((DIM, dk)), full((1, dk)), full((1, 1)), full((dk, dk)), full((1, dk)),
        ],
        out_specs=[
            pl.BlockSpec((BB1, dk), lambda i: (i, 0)),
            pl.BlockSpec((BB1, dk), lambda i: (i, 0)),
            pl.BlockSpec((1, dk), lambda i: (0, 0)),
            pl.BlockSpec((1, dk), lambda i: (0, 0)),
        ],
        out_shape=[
            jax.ShapeDtypeStruct((B, dk), jnp.float32),
            jax.ShapeDtypeStruct((B, dk), jnp.float32),
            jax.ShapeDtypeStruct((1, dk), jnp.float32),
            jax.ShapeDtypeStruct((1, dk), jnp.float32),
        ],
    )(img_rows, sum_t, sum_n, wia, bia, wm, bm,
      w1p, b1p, a1, w1o_p, b1o_p, w2p, b2p, a2, w2o, b2o)


BB2 = 512


def _apply_body(m1_ref, m2_ref, acc1_ref, acc2_ref, img_ref, wia_ref, bia_ref,
                out_ref):
    f32 = jnp.float32
    inv_b = jnp.float32(1.0 / B)
    m1 = m1_ref[...] + acc1_ref[...] * inv_b
    m2 = m2_ref[...] + acc2_ref[...] * inv_b

    w1 = []
    for k in range(K):
        sl = m1[:, k * DIM:(k + 1) * DIM]
        mx = jnp.max(sl, axis=-1, keepdims=True)
        e = jnp.exp(sl - mx)
        w1.append(e / jnp.sum(e, axis=-1, keepdims=True))

    s2 = [m2[:, k * DIM:(k + 1) * DIM] for k in range(K)]
    mx2 = jnp.maximum(jnp.maximum(s2[0], s2[1]), jnp.maximum(s2[2], s2[3]))
    e2 = [jnp.exp(x - mx2) for x in s2]
    den = e2[0] + e2[1] + e2[2] + e2[3]
    w2 = [x / den for x in e2]

    img = img_ref[...].astype(f32)
    auxi = jnp.dot(img, wia_ref[...], preferred_element_type=f32) + bia_ref[...]
    auxi3 = auxi.reshape(BB2, PAD_L, DIM)

    t3 = jnp.zeros((BB2, PAD_L, DIM), f32)
    for k in range(K):
        t1 = jnp.sum(auxi3 * w1[k][:, None, :], axis=-1)
        t3 = t3 + t1[:, :, None] * w2[k][:, None, :]
    out_ref[...] = t3[:, :L, :]


def _tc_apply(m1, m2, acc1, acc2, img_rows, wia, bia):
    dk = DIM * K
    return pl.pallas_call(
        _apply_body,
        grid=(B // BB2,),
        in_specs=[
            pl.BlockSpec((BB2, dk), lambda i: (i, 0)),
            pl.BlockSpec((BB2, dk), lambda i: (i, 0)),
            pl.BlockSpec((1, dk), lambda i: (0, 0)),
            pl.BlockSpec((1, dk), lambda i: (0, 0)),
            pl.BlockSpec((BB2 * PAD_L, DIM_IT), lambda i: (i, 0)),
            pl.BlockSpec((DIM_IT, DIM), lambda i: (0, 0)),
            pl.BlockSpec((1, DIM), lambda i: (0, 0)),
        ],
        out_specs=pl.BlockSpec((BB2, L, DIM), lambda i: (i, 0, 0)),
        out_shape=jax.ShapeDtypeStruct((B, L, DIM), jnp.float32),
    )(m1, m2, acc1, acc2, img_rows, wia, bia)


def kernel(emb, img_emb, W_img2id, b_img2id, W_meta, b_meta,
           W1_pre, b1_pre, a1, W1_out, b1_out,
           W2_pre, b2_pre, a2, W2_out, b2_out,
           item_seq, neighbor_seq):
    dk = DIM * K
    it32 = item_seq.astype(jnp.int32)
    it_idx = it32.reshape(NW, NCH, CL)
    it_idx_pad = jnp.pad(it32, ((0, 0), (0, PAD_L - L))).reshape(NW, NCH, CLP)
    nb_idx = neighbor_seq.astype(jnp.int32).reshape(NW, NCH, CL)
    img_bf = img_emb.astype(jnp.bfloat16)

    sum_t, sum_n = _sc_emb(emb, it_idx, nb_idx)
    img_rows = _sc_img(img_bf, it_idx_pad)

    w1o_p = W1_out.reshape(dk, DIM, K).transpose(0, 2, 1).reshape(dk, dk)
    b1o_p = b1_out.reshape(DIM, K).T.reshape(1, dk)

    m1, m2, acc1, acc2 = _tc_meta(
        img_rows, sum_t, sum_n,
        W_img2id, b_img2id.reshape(1, DIM), W_meta, b_meta.reshape(1, DIM),
        W1_pre, b1_pre.reshape(1, dk), a1.reshape(1, 1), w1o_p, b1o_p,
        W2_pre, b2_pre.reshape(1, dk), a2.reshape(1, 1), W2_out, b2_out.reshape(1, dk))

    return _tc_apply(m1, m2, acc1, acc2, img_rows,
                     W_img2id, b_img2id.reshape(1, DIM))


# speedup vs baseline: 1.0040x; 1.0009x over previous
import jax
import jax.numpy as jnp
from jax import lax
from jax.experimental import pallas as pl
from jax.experimental.pallas import tpu as pltpu
from jax.experimental.pallas import tpu_sc as plsc

NUM_NODE = 100000
DIM = 128
K = 4
DIM_IT = 64
B = 4096
L = 20
PAD_L = 24

NC = 2
NS = 16
NW = NC * NS
BPW = B // NW
CB = 4
CL = CB * L
CLP = CB * PAD_L
NCH = BPW // CB
CPW = 2
WB = CB * CPW
NWV = BPW // WB
RW = CL * CPW


def _sc_emb_body(emb_hbm, it_idx_hbm, nb_idx_hbm,
                 sum_t_hbm, sum_n_hbm,
                 it_idx_v, nb_idx_v,
                 be0, be1, bn0, bn1, ws0, ws1, wn0, wn1,
                 se0, se1, ss0, ss1, sn0, sn1, sv0, sv1):
    c = lax.axis_index("c")
    s = lax.axis_index("s")
    w = c * NS + s
    base_b = w * BPW

    pltpu.sync_copy(it_idx_hbm.at[w], it_idx_v)
    pltpu.sync_copy(nb_idx_hbm.at[w], nb_idx_v)

    def fire(idx_v, wv, buf, se):
        for q in range(CPW):
            pltpu.async_copy(emb_hbm.at[idx_v.at[wv * CPW + q]],
                             buf.at[pl.ds(q * CL, CL)], se)

    def drain(idx_v, wv, buf, se):
        for q in range(CPW):
            pltpu.make_async_copy(emb_hbm.at[idx_v.at[wv * CPW + q]],
                                  buf.at[pl.ds(q * CL, CL)], se).wait()

    def process(idx_v, sum_hbm, wv, buf, se, ws, ss):
        drain(idx_v, wv, buf, se)

        @pl.when(wv >= 2)
        def _():
            pltpu.make_async_copy(ws, sum_hbm.at[pl.ds(base_b, WB)], ss).wait()

        def bb(bi, cc):
            r0 = bi * L
            accs = [buf[r0, pl.ds(g * 16, 16)] for g in range(8)]
            for l in range(1, L):
                for g in range(8):
                    accs[g] = accs[g] + buf[r0 + l, pl.ds(g * 16, 16)]
            for g in range(8):
                ws[bi, pl.ds(g * 16, 16)] = accs[g]
            return cc

        lax.fori_loop(0, WB, bb, 0)
        pltpu.async_copy(ws, sum_hbm.at[pl.ds(base_b + wv * WB, WB)], ss)

        @pl.when(wv + 2 < NWV)
        def _():
            fire(idx_v, wv + 2, buf, se)

    fire(it_idx_v, 0, be0, se0)
    fire(nb_idx_v, 0, bn0, sn0)
    fire(it_idx_v, 1, be1, se1)
    fire(nb_idx_v, 1, bn1, sn1)

    def body(h, carry):
        for par, bt, se, wst, sst, bn, sn, wsn, ssn in (
                (0, be0, se0, ws0, ss0, bn0, sn0, wn0, sv0),
                (1, be1, se1, ws1, ss1, bn1, sn1, wn1, sv1)):
            wv = 2 * h + par
            process(it_idx_v, sum_t_hbm, wv, bt, se, wst, sst)
            process(nb_idx_v, sum_n_hbm, wv, bn, sn, wsn, ssn)
        return carry

    lax.fori_loop(0, NWV // 2, body, 0)
    pltpu.make_async_copy(ws0, sum_t_hbm.at[pl.ds(base_b, WB)], ss0).wait()
    pltpu.make_async_copy(ws1, sum_t_hbm.at[pl.ds(base_b, WB)], ss1).wait()
    pltpu.make_async_copy(wn0, sum_n_hbm.at[pl.ds(base_b, WB)], sv0).wait()
    pltpu.make_async_copy(wn1, sum_n_hbm.at[pl.ds(base_b, WB)], sv1).wait()


def _sc_emb(emb, it_idx, nb_idx):
    mesh = plsc.VectorSubcoreMesh(core_axis_name="c", subcore_axis_name="s")
    f = pl.kernel(
        _sc_emb_body,
        out_type=[
            jax.ShapeDtypeStruct((B, DIM), jnp.float32),
            jax.ShapeDtypeStruct((B, DIM), jnp.float32),
        ],
        mesh=mesh,
        scratch_types=[
            pltpu.VMEM((NCH, CL), jnp.int32),
            pltpu.VMEM((NCH, CL), jnp.int32),
            pltpu.VMEM((RW, DIM), jnp.float32),
            pltpu.VMEM((RW, DIM), jnp.float32),
            pltpu.VMEM((RW, DIM), jnp.float32),
            pltpu.VMEM((RW, DIM), jnp.float32),
            pltpu.VMEM((WB, DIM), jnp.float32),
            pltpu.VMEM((WB, DIM), jnp.float32),
            pltpu.VMEM((WB, DIM), jnp.float32),
            pltpu.VMEM((WB, DIM), jnp.float32),
            pltpu.SemaphoreType.DMA,
            pltpu.SemaphoreType.DMA,
            pltpu.SemaphoreType.DMA,
            pltpu.SemaphoreType.DMA,
            pltpu.SemaphoreType.DMA,
            pltpu.SemaphoreType.DMA,
            pltpu.SemaphoreType.DMA,
            pltpu.SemaphoreType.DMA,
        ],
    )
    return f(emb, it_idx, nb_idx)


def _sc_img_body(img_hbm, it_idx_pad_hbm, img_rows_hbm,
                 it_idx_pad_v, bi0, bi1,
                 si0, si1, sw0, sw1):
    c = lax.axis_index("c")
    s = lax.axis_index("s")
    w = c * NS + s
    base_b = w * BPW

    pltpu.sync_copy(it_idx_pad_hbm.at[w], it_idx_pad_v)

    def img_dst(ci):
        return img_rows_hbm.at[pl.ds((base_b + ci * CB) * PAD_L, CLP)]

    def img_fire(ci, buf, si):
        pltpu.async_copy(img_hbm.at[it_idx_pad_v.at[ci]], buf, si)

    img_fire(0, bi0, si0)
    img_fire(1, bi1, si1)

    def img_body(h, carry):
        for par, buf, si, sw in ((0, bi0, si0, sw0), (1, bi1, si1, sw1)):
            ci = 2 * h + par
            pltpu.make_async_copy(img_hbm.at[it_idx_pad_v.at[ci]], buf, si).wait()
            pltpu.async_copy(buf, img_dst(ci), sw)

            @pl.when(ci + 2 < NCH)
            def _():
                pltpu.make_async_copy(buf, img_dst(ci), sw).wait()
                img_fire(ci + 2, buf, si)
        return carry

    lax.fori_loop(0, NCH // 2, img_body, 0)
    pltpu.make_async_copy(bi0, img_dst(NCH - 2), sw0).wait()
    pltpu.make_async_copy(bi1, img_dst(NCH - 1), sw1).wait()


def _sc_img(img_bf, it_idx_pad):
    mesh = plsc.VectorSubcoreMesh(core_axis_name="c", subcore_axis_name="s")
    f = pl.kernel(
        _sc_img_body,
        out_type=[
            jax.ShapeDtypeStruct((B * PAD_L, DIM_IT), jnp.bfloat16),
        ],
        mesh=mesh,
        scratch_types=[
            pltpu.VMEM((NCH, CLP), jnp.int32),
            pltpu.VMEM((CLP, DIM_IT), jnp.bfloat16),
            pltpu.VMEM((CLP, DIM_IT), jnp.bfloat16),
            pltpu.SemaphoreType.DMA,
            pltpu.SemaphoreType.DMA,
            pltpu.SemaphoreType.DMA,
            pltpu.SemaphoreType.DMA,
        ],
        compiler_params=pltpu.CompilerParams(use_tc_tiling_on_sc=False),
    )
    return f(img_bf, it_idx_pad)[0]


BB1 = 512


def _meta_body(img_ref, sum_t_ref, sum_n_ref,
               wia_ref, bia_ref, wm_ref, bm_ref,
               w1p_ref, b1p_ref, a1_ref, w1o_ref, b1o_ref,
               w2p_ref, b2p_ref, a2_ref, w2o_ref, b2o_ref,
               m1_ref, m2_ref, acc1_ref, acc2_ref):
    f32 = jnp.float32
    img3 = img_ref[...].astype(f32).reshape(BB1, PAD_L, DIM_IT)
    sum_img = jnp.sum(img3[:, :L, :], axis=1)
    sum_auxi = (jnp.dot(sum_img, wia_ref[...], preferred_element_type=f32)
                + jnp.float32(L) * bia_ref[...])
    sum_t = sum_t_ref[...].astype(f32)
    sum_n = sum_n_ref[...].astype(f32)
    tembedi = (jnp.dot(sum_auxi, wm_ref[0:DIM, :], preferred_element_type=f32)
               + jnp.dot(sum_t, wm_ref[DIM:2 * DIM, :], preferred_element_type=f32)
               + jnp.dot(sum_n, wm_ref[2 * DIM:3 * DIM, :], preferred_element_type=f32)
               + bm_ref[...])

    def mlp(wp_ref, bp_ref, a_ref, wo_ref, bo_ref):
        h = jnp.dot(tembedi, wp_ref[...], preferred_element_type=f32) + bp_ref[...]
        a = a_ref[0, 0]
        h = jnp.where(h >= 0, h, a * h)
        m = jnp.dot(h, wo_ref[...], preferred_element_type=f32) + bo_ref[...]
        n = jnp.sqrt(jnp.sum(m * m, axis=-1, keepdims=True))
        return m / jnp.maximum(n, 1e-12)

    m1 = mlp(w1p_ref, b1p_ref, a1_ref, w1o_ref, b1o_ref)
    m2 = mlp(w2p_ref, b2p_ref, a2_ref, w2o_ref, b2o_ref)
    m1_ref[...] = m1
    m2_ref[...] = m2

    @pl.when(pl.program_id(0) == 0)
    def _():
        acc1_ref[...] = jnp.zeros_like(acc1_ref)
        acc2_ref[...] = jnp.zeros_like(acc2_ref)

    acc1_ref[...] += jnp.sum(m1, axis=0, keepdims=True)
    acc2_ref[...] += jnp.sum(m2, axis=0, keepdims=True)


def _tc_meta(img_rows, sum_t, sum_n, wia, bia, wm, bm,
             w1p, b1p, a1, w1o_p, b1o_p, w2p, b2p, a2, w2o, b2o):
    dk = DIM * K
    full = lambda shape: pl.BlockSpec(shape, lambda i: tuple(0 for _ in shape))
    return pl.pallas_call(
        _meta_body,
        grid=(B // BB1,),
        in_specs=[
            pl.BlockSpec((BB1 * PAD_L, DIM_IT), lambda i: (i, 0)),
            pl.BlockSpec((BB1, DIM), lambda i: (i, 0)),
            pl.BlockSpec((BB1, DIM), lambda i: (i, 0)),
            full((DIM_IT, DIM)), full((1, DIM)), full((3 * DIM, DIM)), full((1, DIM)),
            full((DIM, dk)), full((1, dk)), full((1, 1)), full((dk, dk)), full((1, dk)),
            full((DIM, dk)), full((1, dk)), full((1, 1)), full((dk, dk)), full((1, dk)),
        ],
        out_specs=[
            pl.BlockSpec((BB1, dk), lambda i: (i, 0)),
            pl.BlockSpec((BB1, dk), lambda i: (i, 0)),
            pl.BlockSpec((1, dk), lambda i: (0, 0)),
            pl.BlockSpec((1, dk), lambda i: (0, 0)),
        ],
        out_shape=[
            jax.ShapeDtypeStruct((B, dk), jnp.float32),
            jax.ShapeDtypeStruct((B, dk), jnp.float32),
            jax.ShapeDtypeStruct((1, dk), jnp.float32),
            jax.ShapeDtypeStruct((1, dk), jnp.float32),
        ],
    )(img_rows, sum_t, sum_n, wia, bia, wm, bm,
      w1p, b1p, a1, w1o_p, b1o_p, w2p, b2p, a2, w2o, b2o)


BB2 = 512


def _apply_body(m1_ref, m2_ref, acc1_ref, acc2_ref, img_ref, wia_ref, bia_ref,
                out_ref):
    f32 = jnp.float32
    inv_b = jnp.float32(1.0 / B)
    m1 = m1_ref[...] + acc1_ref[...] * inv_b
    m2 = m2_ref[...] + acc2_ref[...] * inv_b

    w1 = []
    for k in range(K):
        sl = m1[:, k * DIM:(k + 1) * DIM]
        mx = jnp.max(sl, axis=-1, keepdims=True)
        e = jnp.exp(sl - mx)
        w1.append(e / jnp.sum(e, axis=-1, keepdims=True))

    s2 = [m2[:, k * DIM:(k + 1) * DIM] for k in range(K)]
    mx2 = jnp.maximum(jnp.maximum(s2[0], s2[1]), jnp.maximum(s2[2], s2[3]))
    e2 = [jnp.exp(x - mx2) for x in s2]
    den = e2[0] + e2[1] + e2[2] + e2[3]
    w2 = [x / den for x in e2]

    img = img_ref[...].astype(f32)
    auxi = jnp.dot(img, wia_ref[...], preferred_element_type=f32) + bia_ref[...]
    auxi3 = auxi.reshape(BB2, PAD_L, DIM)

    t3 = jnp.zeros((BB2, PAD_L, DIM), f32)
    for k in range(K):
        t1 = jnp.sum(auxi3 * w1[k][:, None, :], axis=-1)
        t3 = t3 + t1[:, :, None] * w2[k][:, None, :]
    out_ref[...] = t3[:, :L, :]


def _tc_apply(m1, m2, acc1, acc2, img_rows, wia, bia):
    dk = DIM * K
    return pl.pallas_call(
        _apply_body,
        grid=(B // BB2,),
        in_specs=[
            pl.BlockSpec((BB2, dk), lambda i: (i, 0)),
            pl.BlockSpec((BB2, dk), lambda i: (i, 0)),
            pl.BlockSpec((1, dk), lambda i: (0, 0)),
            pl.BlockSpec((1, dk), lambda i: (0, 0)),
            pl.BlockSpec((BB2 * PAD_L, DIM_IT), lambda i: (i, 0)),
            pl.BlockSpec((DIM_IT, DIM), lambda i: (0, 0)),
            pl.BlockSpec((1, DIM), lambda i: (0, 0)),
        ],
        out_specs=pl.BlockSpec((BB2, L, DIM), lambda i: (i, 0, 0)),
        out_shape=jax.ShapeDtypeStruct((B, L, DIM), jnp.float32),
    )(m1, m2, acc1, acc2, img_rows, wia, bia)


def kernel(emb, img_emb, W_img2id, b_img2id, W_meta, b_meta,
           W1_pre, b1_pre, a1, W1_out, b1_out,
           W2_pre, b2_pre, a2, W2_out, b2_out,
           item_seq, neighbor_seq):
    dk = DIM * K
    it32 = item_seq.astype(jnp.int32)
    it_idx = it32.reshape(NW, NCH, CL)
    it_idx_pad = jnp.pad(it32, ((0, 0), (0, PAD_L - L))).reshape(NW, NCH, CLP)
    nb_idx = neighbor_seq.astype(jnp.int32).reshape(NW, NCH, CL)
    img_bf = img_emb.astype(jnp.bfloat16)

    sum_t, sum_n = _sc_emb(emb, it_idx, nb_idx)
    img_rows = _sc_img(img_bf, it_idx_pad)

    w1o_p = W1_out.reshape(dk, DIM, K).transpose(0, 2, 1).reshape(dk, dk)
    b1o_p = b1_out.reshape(DIM, K).T.reshape(1, dk)

    m1, m2, acc1, acc2 = _tc_meta(
        img_rows, sum_t, sum_n,
        W_img2id, b_img2id.reshape(1, DIM), W_meta, b_meta.reshape(1, DIM),
        W1_pre, b1_pre.reshape(1, dk), a1.reshape(1, 1), w1o_p, b1o_p,
        W2_pre, b2_pre.reshape(1, dk), a2.reshape(1, 1), W2_out, b2_out.reshape(1, dk))

    return _tc_apply(m1, m2, acc1, acc2, img_rows,
                     W_img2id, b_img2id.reshape(1, DIM))
